# baseline (speedup 1.0000x reference)
import functools

import jax
import jax.numpy as jnp
from jax import lax
from jax.experimental import pallas as pl
from jax.experimental.pallas import tpu as pltpu
from jax.experimental.pallas import tpu_sc as plsc

NUM_WORKERS = 32
TOTAL = 32768
PER_W = TOTAL // NUM_WORKERS
CHUNKS = 8
CHUNK = PER_W // CHUNKS
DIM = 32
GROUPS = PER_W // 16


def _vsqrt(x):
    b = lax.bitcast_convert_type(x, jnp.int32)
    y = lax.bitcast_convert_type(
        jnp.int32(0x1FBD1DF5) + lax.shift_right_logical(b, 1), jnp.float32)
    for _ in range(3):
        y = 0.5 * (y + x / y)
    return y


def _make_kernel():
    mesh = plsc.VectorSubcoreMesh(core_axis_name="c", subcore_axis_name="s")

    @functools.partial(
        pl.kernel,
        mesh=mesh,
        out_type=jax.ShapeDtypeStruct((TOTAL,), jnp.float32),
        compiler_params=pltpu.CompilerParams(
            use_tc_tiling_on_sc=False, needs_layout_passes=False),
        scratch_types=[
            pltpu.VMEM((CHUNKS, CHUNK), jnp.int32),
            pltpu.VMEM((CHUNKS, CHUNK), jnp.int32),
            pltpu.VMEM((CHUNKS, CHUNK), jnp.int32),
            pltpu.VMEM((PER_W, DIM), jnp.float32),
            pltpu.VMEM((PER_W, DIM), jnp.float32),
            pltpu.VMEM((PER_W, DIM), jnp.float32),
            pltpu.VMEM((PER_W,), jnp.float32),
            pltpu.SemaphoreType.DMA,
        ],
    )
    def kern(h_idx_hbm, r_idx_hbm, t_idx_hbm, ent_hbm, rel_hbm, out_hbm,
             h_idx, r_idx, t_idx, h_rows, r_rows, t_rows, out_v, sem):
        wid = lax.axis_index("s") * 2 + lax.axis_index("c")

        pltpu.sync_copy(h_idx_hbm.at[wid], h_idx)
        pltpu.sync_copy(r_idx_hbm.at[wid], r_idx)
        pltpu.sync_copy(t_idx_hbm.at[wid], t_idx)

        copies = []
        for j in range(CHUNKS):
            dst = h_rows.at[pl.ds(j * CHUNK, CHUNK)]
            copies.append(pltpu.async_copy(ent_hbm.at[h_idx.at[j]], dst, sem))
        for j in range(CHUNKS):
            dst = t_rows.at[pl.ds(j * CHUNK, CHUNK)]
            copies.append(pltpu.async_copy(ent_hbm.at[t_idx.at[j]], dst, sem))
        for j in range(CHUNKS):
            dst = r_rows.at[pl.ds(j * CHUNK, CHUNK)]
            copies.append(pltpu.async_copy(rel_hbm.at[r_idx.at[j]], dst, sem))
        for c in copies:
            c.wait()

        lane = lax.iota(jnp.int32, 16)

        def group(g, _):
            rows16 = g * 16 + lane
            acc = jnp.zeros((16,), jnp.float32)
            for j in range(DIM):
                cj = jnp.full((16,), j, jnp.int32)
                h = plsc.load_gather(h_rows, [rows16, cj])
                r = plsc.load_gather(r_rows, [rows16, cj])
                t = plsc.load_gather(t_rows, [rows16, cj])
                d = (h + r) - t
                acc = acc + d * d
            out_v[pl.ds(g * 16, 16)] = _vsqrt(acc)
            return 0

        lax.fori_loop(0, GROUPS, group, 0)

        pltpu.sync_copy(out_v, out_hbm.at[pl.ds(wid * PER_W, PER_W)])

    return kern


_KERNEL = _make_kernel()


def kernel(pos_h, pos_r, pos_t, neg_h, neg_r, neg_t, entity_emb, relation_emb):
    h_idx = jnp.concatenate([pos_h, neg_h]).astype(jnp.int32)
    r_idx = jnp.concatenate([pos_r, neg_r]).astype(jnp.int32)
    t_idx = jnp.concatenate([pos_t, neg_t]).astype(jnp.int32)
    shape = (NUM_WORKERS, CHUNKS, CHUNK)
    return _KERNEL(h_idx.reshape(shape), r_idx.reshape(shape),
                   t_idx.reshape(shape), entity_emb, relation_emb)
